# initial kernel scaffold (unmeasured)
import jax
import jax.numpy as jnp
from jax import lax
from jax.experimental import pallas as pl
from jax.experimental.pallas import tpu as pltpu

N_DEV = 8


def kernel(x, w_mat, scale_x, scale_w):
    m_full, k_shard = x.shape
    k_full, n = w_mat.shape
    m_blk = m_full // N_DEV
    assert k_full == N_DEV * k_shard

    def body(x_ref, w_ref, sx_ref, sw_ref, out_ref,
             gath_ref, loc_sem, send_sems, recv_sems):
        my = lax.axis_index("i")

        barrier_sem = pltpu.get_barrier_semaphore()
        for d in range(1, N_DEV):
            peer = lax.rem(my + d, N_DEV)
            pl.semaphore_signal(barrier_sem, inc=1, device_id=(peer,),
                                device_id_type=pl.DeviceIdType.MESH)
        pl.semaphore_wait(barrier_sem, N_DEV - 1)

        local = pltpu.make_async_copy(
            x_ref.at[pl.ds(my * m_blk, m_blk), :], gath_ref.at[my], loc_sem)
        local.start()

        sends = []
        for d in range(1, N_DEV):
            dst = lax.rem(my + d, N_DEV)
            rdma = pltpu.make_async_remote_copy(
                src_ref=x_ref.at[pl.ds(dst * m_blk, m_blk), :],
                dst_ref=gath_ref.at[my],
                send_sem=send_sems.at[d - 1],
                recv_sem=recv_sems.at[my],
                device_id=(dst,),
                device_id_type=pl.DeviceIdType.MESH,
            )
            rdma.start()
            sends.append(rdma)

        local.wait()
        for s in sends:
            s.wait_send()

        for d in range(1, N_DEV):
            src_dev = lax.rem(my + N_DEV - d, N_DEV)
            recv = pltpu.make_async_remote_copy(
                src_ref=x_ref.at[pl.ds(0, m_blk), :],
                dst_ref=gath_ref.at[src_dev],
                send_sem=send_sems.at[0],
                recv_sem=recv_sems.at[src_dev],
                device_id=(src_dev,),
                device_id_type=pl.DeviceIdType.MESH,
            )
            recv.wait_recv()

        out_ref[:, :] = jnp.dot(gath_ref[0], w_ref[0:k_shard, :],
                                preferred_element_type=jnp.float32)
        for j in range(1, N_DEV):
            out_ref[:, :] += jnp.dot(
                gath_ref[j], w_ref[j * k_shard:(j + 1) * k_shard, :],
                preferred_element_type=jnp.float32)
        out_ref[:, :] *= sx_ref[0] * sw_ref[0]

    return pl.pallas_call(
        body,
        out_shape=jax.ShapeDtypeStruct((m_blk, n), jnp.float32),
        in_specs=[
            pl.BlockSpec(memory_space=pltpu.VMEM),
            pl.BlockSpec(memory_space=pltpu.VMEM),
            pl.BlockSpec(memory_space=pltpu.SMEM),
            pl.BlockSpec(memory_space=pltpu.SMEM),
        ],
        out_specs=pl.BlockSpec(memory_space=pltpu.VMEM),
        scratch_shapes=[
            pltpu.VMEM((N_DEV, m_blk, k_shard), x.dtype),
            pltpu.SemaphoreType.DMA,
            pltpu.SemaphoreType.DMA((N_DEV - 1,)),
            pltpu.SemaphoreType.DMA((N_DEV,)),
        ],
        compiler_params=pltpu.CompilerParams(collective_id=0),
    )(x, w_mat, scale_x, scale_w)


# baseline (device time: 99900 ns/iter reference)
import jax
import jax.numpy as jnp
from jax import lax
from jax.experimental import pallas as pl
from jax.experimental.pallas import tpu as pltpu

N_DEV = 8


def kernel(x, w_mat, scale_x, scale_w):
    m_full, k_shard = x.shape
    k_full, n = w_mat.shape
    m_blk = m_full // N_DEV
    assert k_full == N_DEV * k_shard

    def body(x_ref, w_ref, sx_ref, sw_ref, out_ref,
             gath_ref, loc_sem, send_sems, recv_sems):
        my = lax.axis_index("i")

        barrier_sem = pltpu.get_barrier_semaphore()
        for d in range(1, N_DEV):
            peer = lax.rem(my + d, N_DEV)
            pl.semaphore_signal(barrier_sem, inc=1, device_id=(peer,),
                                device_id_type=pl.DeviceIdType.MESH)
        pl.semaphore_wait(barrier_sem, N_DEV - 1)

        local = pltpu.make_async_copy(
            x_ref.at[pl.ds(my * m_blk, m_blk), :], gath_ref.at[my], loc_sem)
        local.start()

        sends = []
        for d in range(1, N_DEV):
            dst = lax.rem(my + d, N_DEV)
            rdma = pltpu.make_async_remote_copy(
                src_ref=x_ref.at[pl.ds(dst * m_blk, m_blk), :],
                dst_ref=gath_ref.at[my],
                send_sem=send_sems.at[d - 1],
                recv_sem=recv_sems.at[my],
                device_id=(dst,),
                device_id_type=pl.DeviceIdType.MESH,
            )
            rdma.start()
            sends.append(rdma)

        local.wait()
        for s in sends:
            s.wait_send()

        for d in range(1, N_DEV):
            src_dev = lax.rem(my + N_DEV - d, N_DEV)
            recv = pltpu.make_async_remote_copy(
                src_ref=x_ref.at[pl.ds(0, m_blk), :],
                dst_ref=gath_ref.at[src_dev],
                send_sem=send_sems.at[0],
                recv_sem=recv_sems.at[src_dev],
                device_id=(src_dev,),
                device_id_type=pl.DeviceIdType.MESH,
            )
            recv.wait_recv()

        out_ref[:, :] = jnp.dot(gath_ref[0], w_ref[0:k_shard, :],
                                preferred_element_type=jnp.float32)
        for j in range(1, N_DEV):
            out_ref[:, :] += jnp.dot(
                gath_ref[j], w_ref[j * k_shard:(j + 1) * k_shard, :],
                preferred_element_type=jnp.float32)
        out_ref[:, :] *= sx_ref[0] * sw_ref[0]

    return pl.pallas_call(
        body,
        out_shape=jax.ShapeDtypeStruct((m_blk, n), jnp.float32),
        in_specs=[
            pl.BlockSpec(memory_space=pltpu.VMEM),
            pl.BlockSpec(memory_space=pltpu.VMEM),
            pl.BlockSpec(memory_space=pltpu.SMEM),
            pl.BlockSpec(memory_space=pltpu.SMEM),
        ],
        out_specs=pl.BlockSpec(memory_space=pltpu.VMEM),
        scratch_shapes=[
            pltpu.VMEM((N_DEV, m_blk, k_shard), x.dtype),
            pltpu.SemaphoreType.DMA,
            pltpu.SemaphoreType.DMA((N_DEV - 1,)),
            pltpu.SemaphoreType.DMA((N_DEV,)),
        ],
        compiler_params=pltpu.CompilerParams(
            collective_id=0, vmem_limit_bytes=100 * 1024 * 1024),
    )(x, w_mat, scale_x, scale_w)


# device time: 41320 ns/iter; 2.4177x vs baseline; 2.4177x over previous
import jax
import jax.numpy as jnp
from jax import lax
from jax.experimental import pallas as pl
from jax.experimental.pallas import tpu as pltpu

N_DEV = 8
E4M3 = jnp.float8_e4m3fn
E5M2 = jnp.float8_e5m2


def _dot(a, b):
    return lax.dot_general(a, b, (((1,), (0,)), ((), ())),
                           preferred_element_type=jnp.float32)


def kernel(x, w_mat, scale_x, scale_w):
    m_full, k_shard = x.shape
    k_full, n = w_mat.shape
    m_blk = m_full // N_DEV
    assert k_full == N_DEV * k_shard

    def body(x_ref, w_ref, sx_ref, sw_ref, out_ref,
             stage_ref, gath_ref, w8_ref, loc_sem, send_sems, recv_sems):
        my = lax.axis_index("i")

        barrier_sem = pltpu.get_barrier_semaphore()
        for d in range(1, N_DEV):
            peer = lax.rem(my + d, N_DEV)
            pl.semaphore_signal(barrier_sem, inc=1, device_id=(peer,),
                                device_id_type=pl.DeviceIdType.MESH)
        pl.semaphore_wait(barrier_sem, N_DEV - 1)

        stage_ref[...] = x_ref[...].reshape(N_DEV, m_blk, k_shard).astype(E4M3)

        sends = []
        for d in range(1, N_DEV):
            dst = lax.rem(my + d, N_DEV)
            rdma = pltpu.make_async_remote_copy(
                src_ref=stage_ref.at[dst],
                dst_ref=gath_ref.at[my],
                send_sem=send_sems.at[d - 1],
                recv_sem=recv_sems.at[my],
                device_id=(dst,),
                device_id_type=pl.DeviceIdType.MESH,
            )
            rdma.start()
            sends.append(rdma)

        local = pltpu.make_async_copy(stage_ref.at[my], gath_ref.at[my],
                                      loc_sem)
        local.start()

        w8_ref[...] = w_ref[...].astype(E5M2)

        local.wait()
        out_ref[...] = _dot(gath_ref[my],
                            w8_ref[pl.ds(my * k_shard, k_shard), :])

        for d in range(1, N_DEV):
            src_dev = lax.rem(my + N_DEV - d, N_DEV)
            recv = pltpu.make_async_remote_copy(
                src_ref=stage_ref.at[src_dev],
                dst_ref=gath_ref.at[src_dev],
                send_sem=send_sems.at[0],
                recv_sem=recv_sems.at[src_dev],
                device_id=(src_dev,),
                device_id_type=pl.DeviceIdType.MESH,
            )
            recv.wait_recv()
            out_ref[...] += _dot(gath_ref[src_dev],
                                 w8_ref[pl.ds(src_dev * k_shard, k_shard), :])

        out_ref[...] *= sx_ref[0] * sw_ref[0]

        for s in sends:
            s.wait_send()

    return pl.pallas_call(
        body,
        out_shape=jax.ShapeDtypeStruct((m_blk, n), jnp.float32),
        in_specs=[
            pl.BlockSpec(memory_space=pltpu.VMEM),
            pl.BlockSpec(memory_space=pltpu.VMEM),
            pl.BlockSpec(memory_space=pltpu.SMEM),
            pl.BlockSpec(memory_space=pltpu.SMEM),
        ],
        out_specs=pl.BlockSpec(memory_space=pltpu.VMEM),
        scratch_shapes=[
            pltpu.VMEM((N_DEV, m_blk, k_shard), E4M3),
            pltpu.VMEM((N_DEV, m_blk, k_shard), E4M3),
            pltpu.VMEM((k_full, n), E5M2),
            pltpu.SemaphoreType.DMA,
            pltpu.SemaphoreType.DMA((N_DEV - 1,)),
            pltpu.SemaphoreType.DMA((N_DEV,)),
        ],
        compiler_params=pltpu.CompilerParams(
            collective_id=0, vmem_limit_bytes=100 * 1024 * 1024),
    )(x, w_mat, scale_x, scale_w)


# device time: 31688 ns/iter; 3.1526x vs baseline; 1.3040x over previous
import jax
import jax.numpy as jnp
from jax import lax
from jax.experimental import pallas as pl
from jax.experimental.pallas import tpu as pltpu

N_DEV = 8
E4M3 = jnp.float8_e4m3fn
E5M2 = jnp.float8_e5m2


def _dot(a, b):
    return lax.dot_general(a, b, (((1,), (0,)), ((), ())),
                           preferred_element_type=jnp.float32)


def kernel(x, w_mat, scale_x, scale_w):
    m_full, k_shard = x.shape
    k_full, n = w_mat.shape
    m_blk = m_full // N_DEV
    assert k_full == N_DEV * k_shard

    def body(x_ref, w_ref, sx_ref, sw_ref, out_ref,
             stage_ref, gath_ref, wchunks_ref,
             loc_sem, wsems, send_sems, recv_sems):
        my = lax.axis_index("i")

        wdmas = []
        for d in range(N_DEV):
            s = lax.rem(my + N_DEV - d, N_DEV)
            wdma = pltpu.make_async_copy(
                w_ref.at[pl.ds(s * k_shard, k_shard), :],
                wchunks_ref.at[d], wsems.at[d])
            wdma.start()
            wdmas.append(wdma)

        barrier_sem = pltpu.get_barrier_semaphore()
        for d in range(1, N_DEV):
            peer = lax.rem(my + d, N_DEV)
            pl.semaphore_signal(barrier_sem, inc=1, device_id=(peer,),
                                device_id_type=pl.DeviceIdType.MESH)
        pl.semaphore_wait(barrier_sem, N_DEV - 1)

        stage_ref[...] = x_ref[...].reshape(N_DEV, m_blk, k_shard).astype(E4M3)

        sends = []
        for d in range(1, N_DEV):
            dst = lax.rem(my + d, N_DEV)
            rdma = pltpu.make_async_remote_copy(
                src_ref=stage_ref.at[dst],
                dst_ref=gath_ref.at[my],
                send_sem=send_sems.at[d - 1],
                recv_sem=recv_sems.at[my],
                device_id=(dst,),
                device_id_type=pl.DeviceIdType.MESH,
            )
            rdma.start()
            sends.append(rdma)

        local = pltpu.make_async_copy(stage_ref.at[my], gath_ref.at[my],
                                      loc_sem)
        local.start()

        local.wait()
        wdmas[0].wait()
        out_ref[...] = _dot(gath_ref[my], wchunks_ref[0].astype(E5M2))

        for d in range(1, N_DEV):
            src_dev = lax.rem(my + N_DEV - d, N_DEV)
            recv = pltpu.make_async_remote_copy(
                src_ref=stage_ref.at[src_dev],
                dst_ref=gath_ref.at[src_dev],
                send_sem=send_sems.at[0],
                recv_sem=recv_sems.at[src_dev],
                device_id=(src_dev,),
                device_id_type=pl.DeviceIdType.MESH,
            )
            recv.wait_recv()
            wdmas[d].wait()
            out_ref[...] += _dot(gath_ref[src_dev],
                                 wchunks_ref[d].astype(E5M2))

        out_ref[...] *= sx_ref[0] * sw_ref[0]

        for s in sends:
            s.wait_send()

    return pl.pallas_call(
        body,
        out_shape=jax.ShapeDtypeStruct((m_blk, n), jnp.float32),
        in_specs=[
            pl.BlockSpec(memory_space=pltpu.VMEM),
            pl.BlockSpec(memory_space=pl.ANY),
            pl.BlockSpec(memory_space=pltpu.SMEM),
            pl.BlockSpec(memory_space=pltpu.SMEM),
        ],
        out_specs=pl.BlockSpec(memory_space=pltpu.VMEM),
        scratch_shapes=[
            pltpu.VMEM((N_DEV, m_blk, k_shard), E4M3),
            pltpu.VMEM((N_DEV, m_blk, k_shard), E4M3),
            pltpu.VMEM((N_DEV, k_shard, n), jnp.float32),
            pltpu.SemaphoreType.DMA,
            pltpu.SemaphoreType.DMA((N_DEV,)),
            pltpu.SemaphoreType.DMA((N_DEV - 1,)),
            pltpu.SemaphoreType.DMA((N_DEV,)),
        ],
        compiler_params=pltpu.CompilerParams(
            collective_id=0, vmem_limit_bytes=100 * 1024 * 1024),
    )(x, w_mat, scale_x, scale_w)
